# baseline (speedup 1.0000x reference)
import functools

import jax
import jax.numpy as jnp
from jax import lax
from jax.experimental import pallas as pl
from jax.experimental.pallas import tpu as pltpu
from jax.experimental.pallas import tpu_sc as plsc

_NUM_OUT = 10000
_NUM_IN = 320000
_D = 128
_NNZ = 640000

_NC = 2
_NS = 16
_NW = _NC * _NS
_CH = 128
_CHUNKS = 159
_EDGES_PER_W = _CHUNKS * _CH
_NPAD = _EDGES_PER_W * _NW

_ROW_STRIDE = 624
_RCH = 128
_RCOPIES = 5


def _sc_body(feat_hbm, rows_hbm, cols_hbm, vals_hbm, out_hbm,
             cols_v0, cols_v1, cols_v2, vals_v0, vals_v1, vals_v2,
             rows_v0, rows_v1, rows_v2, gbuf0, gbuf1, gbuf2, acc_sh,
             gsem0, gsem1, gsem2, csem0, csem1, csem2,
             vsem0, vsem1, vsem2, rsem0, rsem1, rsem2,
             ssem0, ssem1, ssem2):
    c = lax.axis_index("c")
    s = lax.axis_index("s")
    wid = c * _NS + s
    eb = wid * _EDGES_PER_W

    @plsc.parallel_loop(0, _RCH, 1, unroll=4)
    def _zero_row(e):
        zero = jnp.zeros((16,), jnp.float32)
        for j in range(_D // 16):
            gbuf0[e, pl.ds(j * 16, 16)] = zero
    row0 = s * _ROW_STRIDE
    for i in range(_RCOPIES):
        pltpu.async_copy(gbuf0, acc_sh.at[pl.ds(row0 + i * _RCH, _RCH)],
                         gsem0)
    for i in range(_RCOPIES):
        pltpu.make_async_copy(
            gbuf0, acc_sh.at[pl.ds(row0, _RCH)], gsem0).wait()
    plsc.subcore_barrier()

    gb = (gbuf0, gbuf1, gbuf2)
    gs = (gsem0, gsem1, gsem2)
    cv = (cols_v0, cols_v1, cols_v2)
    cs = (csem0, csem1, csem2)
    vv = (vals_v0, vals_v1, vals_v2)
    vs = (vsem0, vsem1, vsem2)
    rv = (rows_v0, rows_v1, rows_v2)
    rs = (rsem0, rsem1, rsem2)
    ss = (ssem0, ssem1, ssem2)

    for b in range(2):
        pltpu.sync_copy(cols_hbm.at[pl.ds(eb + b * _CH, _CH)], cv[b])
        pltpu.async_copy(vals_hbm.at[pl.ds(eb + b * _CH, _CH)],
                         vv[b].at[pl.ds(0, _CH)], vs[b])
        pltpu.async_copy(rows_hbm.at[pl.ds(eb + b * _CH, _CH)], rv[b], rs[b])
        for h in range(2):
            hs = pl.ds(h * (_CH // 2), _CH // 2)
            pltpu.async_copy(feat_hbm.at[cv[b].at[hs]], gb[b].at[hs], gs[b])

    def _triple(k, _):
        for b in range(3):
            ci = 3 * k + b
            b2 = (b + 2) % 3
            live = ci < _CHUNKS - 2
            with jax.named_scope("gwait"):
                pltpu.make_async_copy(feat_hbm.at[cv[b]], gb[b], gs[b]).wait()

            @pl.when(live)
            def _():
                nc = ci + 2
                pltpu.async_copy(cols_hbm.at[pl.ds(eb + nc * _CH, _CH)],
                                 cv[b2], cs[b2])
                pltpu.async_copy(vals_hbm.at[pl.ds(eb + nc * _CH, _CH)],
                                 vv[b2].at[pl.ds(0, _CH)], vs[b2])

            pltpu.make_async_copy(
                vals_hbm.at[pl.ds(0, _CH)], vv[b].at[pl.ds(0, _CH)],
                vs[b]).wait()

            with jax.named_scope("scale"):
                @plsc.parallel_loop(0, _CH, 1, unroll=4)
                def _scale(e):
                    sc_ = jnp.full((16,), vv[b][pl.ds(e, 16)][0], jnp.float32)
                    for j in range(_D // 16):
                        sl = pl.ds(j * 16, 16)
                        gb[b][e, sl] = gb[b][e, sl] * sc_

            pltpu.make_async_copy(rows_hbm.at[pl.ds(0, _CH)],
                                  rv[b], rs[b]).wait()
            pltpu.async_copy(gb[b], acc_sh.at[rv[b]], ss[b], add=True)

            @pl.when(jnp.logical_and(live, ci >= 1))
            def _():
                with jax.named_scope("swait"):
                    pltpu.make_async_copy(
                        gb[b2], acc_sh.at[rv[b2]], ss[b2]).wait()

            @pl.when(live)
            def _():
                nc = ci + 2
                pltpu.make_async_copy(
                    cols_hbm.at[pl.ds(0, _CH)], cv[b2], cs[b2]).wait()
                for h in range(2):
                    hs = pl.ds(h * (_CH // 2), _CH // 2)
                    pltpu.async_copy(feat_hbm.at[cv[b2].at[hs]],
                                     gb[b2].at[hs], gs[b2])
                pltpu.async_copy(rows_hbm.at[pl.ds(eb + nc * _CH, _CH)],
                                 rv[b2], rs[b2])
        return 0
    lax.fori_loop(0, _CHUNKS // 3, _triple, 0)

    for b in range(3):
        pltpu.make_async_copy(gb[b], acc_sh.at[rv[b]], ss[b]).wait()

    plsc.subcore_barrier()
    for i in range(_RCOPIES):
        r0 = row0 + i * _RCH
        pltpu.async_copy(acc_sh.at[pl.ds(r0, _RCH)],
                         out_hbm.at[c, pl.ds(r0, _RCH)], gsem0)
    for i in range(_RCOPIES):
        pltpu.make_async_copy(acc_sh.at[pl.ds(row0, _RCH)],
                              out_hbm.at[c, pl.ds(row0, _RCH)], gsem0).wait()


_sc_spmm = functools.partial(
    pl.kernel,
    out_type=jax.ShapeDtypeStruct((_NC, _NUM_OUT, _D), jnp.float32),
    mesh=plsc.VectorSubcoreMesh(core_axis_name="c", subcore_axis_name="s"),
    scratch_types=(
        [pltpu.VMEM((_CH,), jnp.int32)] * 3
        + [pltpu.VMEM((_CH + 16,), jnp.float32)] * 3
        + [pltpu.VMEM((_CH,), jnp.int32)] * 3
        + [pltpu.VMEM((_CH, _D), jnp.float32)] * 3
        + [pltpu.VMEM_SHARED((_NUM_OUT, _D), jnp.float32)]
        + [pltpu.SemaphoreType.DMA] * 15
    ),
)(_sc_body)


def _sum2_body(p_ref, o_ref):
    o_ref[...] = p_ref[0] + p_ref[1]


def _sum_partials(partials):
    blk = 1000
    return pl.pallas_call(
        _sum2_body,
        grid=(_NUM_OUT // blk,),
        in_specs=[pl.BlockSpec((_NC, blk, _D), lambda i: (0, i, 0))],
        out_specs=pl.BlockSpec((blk, _D), lambda i: (i, 0)),
        out_shape=jax.ShapeDtypeStruct((_NUM_OUT, _D), jnp.float32),
    )(partials)


@jax.jit
def kernel(simplex_features, boundary_indices, boundary_values):
    rows = boundary_indices[0].astype(jnp.int32)
    cols = boundary_indices[1].astype(jnp.int32)
    vals = boundary_values.astype(jnp.float32)
    pad = _NPAD - _NNZ
    spread = jnp.arange(pad, dtype=jnp.int32)
    rows = jnp.concatenate([rows, spread % _NUM_OUT])
    cols = jnp.concatenate([cols, spread % _NUM_IN])
    vals = jnp.concatenate([vals, jnp.zeros((pad,), jnp.float32)])
    partials = _sc_spmm(simplex_features, rows, cols, vals)
    return _sum_partials(partials)


# speedup vs baseline: 1.0178x; 1.0178x over previous
import functools

import jax
import jax.numpy as jnp
from jax import lax
from jax.experimental import pallas as pl
from jax.experimental.pallas import tpu as pltpu
from jax.experimental.pallas import tpu_sc as plsc

_NUM_OUT = 10000
_NUM_IN = 320000
_D = 128
_NNZ = 640000

_NC = 2
_NS = 16
_NW = _NC * _NS
_CH = 128
_CHUNKS_BASE = 156
_EXTRA_TILES = 5000 - _NW * _CHUNKS_BASE

_ROW_STRIDE = 624
_RCH = 128
_RCOPIES = 5


def _sc_body(feat_hbm, rows_hbm, cols_hbm, vals_hbm, out_hbm,
             cols_v0, cols_v1, cols_v2, vals_v0, vals_v1, vals_v2,
             rows_v0, rows_v1, rows_v2, gbuf0, gbuf1, gbuf2, acc_sh,
             gsem0, gsem1, gsem2, csem0, csem1, csem2,
             vsem0, vsem1, vsem2, rsem0, rsem1, rsem2,
             ssem0, ssem1, ssem2):
    c = lax.axis_index("c")
    s = lax.axis_index("s")
    wid = c * _NS + s
    eb = (wid * _CHUNKS_BASE + jnp.minimum(wid, _EXTRA_TILES)) * _CH
    my_chunks = _CHUNKS_BASE + jnp.where(wid < _EXTRA_TILES, 1, 0)

    @plsc.parallel_loop(0, _RCH, 1, unroll=4)
    def _zero_row(e):
        zero = jnp.zeros((16,), jnp.float32)
        for j in range(_D // 16):
            gbuf0[e, pl.ds(j * 16, 16)] = zero
    row0 = s * _ROW_STRIDE
    for i in range(_RCOPIES):
        pltpu.async_copy(gbuf0, acc_sh.at[pl.ds(row0 + i * _RCH, _RCH)],
                         gsem0)
    for i in range(_RCOPIES):
        pltpu.make_async_copy(
            gbuf0, acc_sh.at[pl.ds(row0, _RCH)], gsem0).wait()
    plsc.subcore_barrier()

    gb = (gbuf0, gbuf1, gbuf2)
    gs = (gsem0, gsem1, gsem2)
    cv = (cols_v0, cols_v1, cols_v2)
    cs = (csem0, csem1, csem2)
    vv = (vals_v0, vals_v1, vals_v2)
    vs = (vsem0, vsem1, vsem2)
    rv = (rows_v0, rows_v1, rows_v2)
    rs = (rsem0, rsem1, rsem2)
    ss = (ssem0, ssem1, ssem2)

    for b in range(2):
        pltpu.sync_copy(cols_hbm.at[pl.ds(eb + b * _CH, _CH)], cv[b])
        pltpu.async_copy(vals_hbm.at[pl.ds(eb + b * _CH, _CH)],
                         vv[b].at[pl.ds(0, _CH)], vs[b])
        pltpu.async_copy(rows_hbm.at[pl.ds(eb + b * _CH, _CH)], rv[b], rs[b])
        for h in range(2):
            hs = pl.ds(h * (_CH // 2), _CH // 2)
            pltpu.async_copy(feat_hbm.at[cv[b].at[hs]], gb[b].at[hs], gs[b])

    def _triple(k, _):
        for b in range(3):
            ci = 3 * k + b
            b2 = (b + 2) % 3
            live = ci < my_chunks - 2
            with jax.named_scope("gwait"):
                pltpu.make_async_copy(feat_hbm.at[cv[b]], gb[b], gs[b]).wait()

            @pl.when(live)
            def _():
                nc = ci + 2
                pltpu.async_copy(cols_hbm.at[pl.ds(eb + nc * _CH, _CH)],
                                 cv[b2], cs[b2])
                pltpu.async_copy(vals_hbm.at[pl.ds(eb + nc * _CH, _CH)],
                                 vv[b2].at[pl.ds(0, _CH)], vs[b2])

            pltpu.make_async_copy(
                vals_hbm.at[pl.ds(0, _CH)], vv[b].at[pl.ds(0, _CH)],
                vs[b]).wait()

            with jax.named_scope("scale"):
                @plsc.parallel_loop(0, _CH, 1, unroll=4)
                def _scale(e):
                    sc_ = jnp.full((16,), vv[b][pl.ds(e, 16)][0], jnp.float32)
                    for j in range(_D // 16):
                        sl = pl.ds(j * 16, 16)
                        gb[b][e, sl] = gb[b][e, sl] * sc_

            pltpu.make_async_copy(rows_hbm.at[pl.ds(0, _CH)],
                                  rv[b], rs[b]).wait()
            pltpu.async_copy(gb[b], acc_sh.at[rv[b]], ss[b], add=True)

            @pl.when(jnp.logical_and(live, ci >= 1))
            def _():
                with jax.named_scope("swait"):
                    pltpu.make_async_copy(
                        gb[b2], acc_sh.at[rv[b2]], ss[b2]).wait()

            @pl.when(live)
            def _():
                nc = ci + 2
                pltpu.make_async_copy(
                    cols_hbm.at[pl.ds(0, _CH)], cv[b2], cs[b2]).wait()
                for h in range(2):
                    hs = pl.ds(h * (_CH // 2), _CH // 2)
                    pltpu.async_copy(feat_hbm.at[cv[b2].at[hs]],
                                     gb[b2].at[hs], gs[b2])
                pltpu.async_copy(rows_hbm.at[pl.ds(eb + nc * _CH, _CH)],
                                 rv[b2], rs[b2])
        return 0
    lax.fori_loop(0, _CHUNKS_BASE // 3, _triple, 0)

    @pl.when(wid < _EXTRA_TILES)
    def _():
        tb = 0
        pltpu.make_async_copy(feat_hbm.at[cv[tb]], gb[tb], gs[tb]).wait()
        pltpu.make_async_copy(
            vals_hbm.at[pl.ds(0, _CH)], vv[tb].at[pl.ds(0, _CH)],
            vs[tb]).wait()

        @plsc.parallel_loop(0, _CH, 1, unroll=4)
        def _scale_tail(e):
            sc_ = jnp.full((16,), vv[tb][pl.ds(e, 16)][0], jnp.float32)
            for j in range(_D // 16):
                sl = pl.ds(j * 16, 16)
                gb[tb][e, sl] = gb[tb][e, sl] * sc_

        pltpu.make_async_copy(rows_hbm.at[pl.ds(0, _CH)],
                              rv[tb], rs[tb]).wait()
        pltpu.async_copy(gb[tb], acc_sh.at[rv[tb]], ss[tb], add=True)

    for b in range(3):
        pltpu.make_async_copy(gb[b], acc_sh.at[rv[b]], ss[b]).wait()

    plsc.subcore_barrier()
    for i in range(_RCOPIES):
        r0 = row0 + i * _RCH
        pltpu.async_copy(acc_sh.at[pl.ds(r0, _RCH)],
                         out_hbm.at[c, pl.ds(r0, _RCH)], gsem0)
    for i in range(_RCOPIES):
        pltpu.make_async_copy(acc_sh.at[pl.ds(row0, _RCH)],
                              out_hbm.at[c, pl.ds(row0, _RCH)], gsem0).wait()


_sc_spmm = functools.partial(
    pl.kernel,
    out_type=jax.ShapeDtypeStruct((_NC, _NUM_OUT, _D), jnp.float32),
    mesh=plsc.VectorSubcoreMesh(core_axis_name="c", subcore_axis_name="s"),
    scratch_types=(
        [pltpu.VMEM((_CH,), jnp.int32)] * 3
        + [pltpu.VMEM((_CH + 16,), jnp.float32)] * 3
        + [pltpu.VMEM((_CH,), jnp.int32)] * 3
        + [pltpu.VMEM((_CH, _D), jnp.float32)] * 3
        + [pltpu.VMEM_SHARED((_NUM_OUT, _D), jnp.float32)]
        + [pltpu.SemaphoreType.DMA] * 15
    ),
)(_sc_body)


def _sum2_body(p_ref, o_ref):
    o_ref[...] = p_ref[0] + p_ref[1]


def _sum_partials(partials):
    blk = 1000
    return pl.pallas_call(
        _sum2_body,
        grid=(_NUM_OUT // blk,),
        in_specs=[pl.BlockSpec((_NC, blk, _D), lambda i: (0, i, 0))],
        out_specs=pl.BlockSpec((blk, _D), lambda i: (i, 0)),
        out_shape=jax.ShapeDtypeStruct((_NUM_OUT, _D), jnp.float32),
    )(partials)


@jax.jit
def kernel(simplex_features, boundary_indices, boundary_values):
    rows = boundary_indices[0].astype(jnp.int32)
    cols = boundary_indices[1].astype(jnp.int32)
    vals = boundary_values.astype(jnp.float32)
    partials = _sc_spmm(simplex_features, rows, cols, vals)
    return _sum_partials(partials)


# speedup vs baseline: 1.0255x; 1.0075x over previous
import functools

import jax
import jax.numpy as jnp
from jax import lax
from jax.experimental import pallas as pl
from jax.experimental.pallas import tpu as pltpu
from jax.experimental.pallas import tpu_sc as plsc

_NUM_OUT = 10000
_NUM_IN = 320000
_D = 128
_NNZ = 640000

_NC = 2
_NS = 16
_NW = _NC * _NS
_CH = 128
_CHUNKS_BASE = 156
_EXTRA_TILES = 5000 - _NW * _CHUNKS_BASE

_ROW_STRIDE = 624
_RCH = 128
_RCOPIES = 5


def _sc_body(feat_hbm, rows_hbm, cols_hbm, vals_hbm, out_hbm,
             cols_v0, cols_v1, cols_v2, vals_v0, vals_v1, vals_v2,
             rows_v0, rows_v1, rows_v2, gbuf0, gbuf1, gbuf2, acc_sh,
             gsem0, gsem1, gsem2, csem0, csem1, csem2,
             vsem0, vsem1, vsem2, rsem0, rsem1, rsem2,
             ssem0, ssem1, ssem2):
    c = lax.axis_index("c")
    s = lax.axis_index("s")
    wid = c * _NS + s
    eb = (wid * _CHUNKS_BASE + jnp.minimum(wid, _EXTRA_TILES)) * _CH
    my_chunks = _CHUNKS_BASE + jnp.where(wid < _EXTRA_TILES, 1, 0)

    @plsc.parallel_loop(0, _RCH, 1, unroll=4)
    def _zero_row(e):
        zero = jnp.zeros((16,), jnp.float32)
        for j in range(_D // 16):
            gbuf0[e, pl.ds(j * 16, 16)] = zero
    row0 = s * _ROW_STRIDE
    for i in range(_RCOPIES):
        pltpu.async_copy(gbuf0, acc_sh.at[pl.ds(row0 + i * _RCH, _RCH)],
                         gsem0)
    for i in range(_RCOPIES):
        pltpu.make_async_copy(
            gbuf0, acc_sh.at[pl.ds(row0, _RCH)], gsem0).wait()
    plsc.subcore_barrier()

    gb = (gbuf0, gbuf1, gbuf2)
    gs = (gsem0, gsem1, gsem2)
    cv = (cols_v0, cols_v1, cols_v2)
    cs = (csem0, csem1, csem2)
    vv = (vals_v0, vals_v1, vals_v2)
    vs = (vsem0, vsem1, vsem2)
    rv = (rows_v0, rows_v1, rows_v2)
    rs = (rsem0, rsem1, rsem2)
    ss = (ssem0, ssem1, ssem2)

    for b in range(2):
        pltpu.sync_copy(cols_hbm.at[pl.ds(eb + b * _CH, _CH)], cv[b])
        pltpu.async_copy(vals_hbm.at[pl.ds(eb + b * _CH, _CH)],
                         vv[b].at[pl.ds(0, _CH)], vs[b])
        pltpu.async_copy(rows_hbm.at[pl.ds(eb + b * _CH, _CH)], rv[b], rs[b])
        for h in range(2):
            hs = pl.ds(h * (_CH // 2), _CH // 2)
            pltpu.async_copy(feat_hbm.at[cv[b].at[hs]], gb[b].at[hs], gs[b])

    def _triple(k, _):
        for b in range(3):
            ci = 3 * k + b
            b2 = (b + 2) % 3
            live = ci < my_chunks - 2
            pltpu.make_async_copy(feat_hbm.at[cv[b]], gb[b], gs[b]).wait()

            @pl.when(live)
            def _():
                nc = ci + 2
                pltpu.async_copy(cols_hbm.at[pl.ds(eb + nc * _CH, _CH)],
                                 cv[b2], cs[b2])
                pltpu.async_copy(vals_hbm.at[pl.ds(eb + nc * _CH, _CH)],
                                 vv[b2].at[pl.ds(0, _CH)], vs[b2])

            pltpu.make_async_copy(
                vals_hbm.at[pl.ds(0, _CH)], vv[b].at[pl.ds(0, _CH)],
                vs[b]).wait()

            @plsc.parallel_loop(0, _CH, 1, unroll=4)
            def _scale(e):
                sc_ = jnp.full((16,), vv[b][pl.ds(e, 16)][0], jnp.float32)
                for j in range(_D // 16):
                    sl = pl.ds(j * 16, 16)
                    gb[b][e, sl] = gb[b][e, sl] * sc_

            pltpu.make_async_copy(rows_hbm.at[pl.ds(0, _CH)],
                                  rv[b], rs[b]).wait()
            pltpu.async_copy(gb[b], acc_sh.at[rv[b]], ss[b], add=True)

            @pl.when(jnp.logical_and(live, ci >= 1))
            def _():
                pltpu.make_async_copy(
                    gb[b2], acc_sh.at[rv[b2]], ss[b2]).wait()

            @pl.when(live)
            def _():
                nc = ci + 2
                pltpu.make_async_copy(
                    cols_hbm.at[pl.ds(0, _CH)], cv[b2], cs[b2]).wait()
                for h in range(2):
                    hs = pl.ds(h * (_CH // 2), _CH // 2)
                    pltpu.async_copy(feat_hbm.at[cv[b2].at[hs]],
                                     gb[b2].at[hs], gs[b2])
                pltpu.async_copy(rows_hbm.at[pl.ds(eb + nc * _CH, _CH)],
                                 rv[b2], rs[b2])
        return 0
    lax.fori_loop(0, _CHUNKS_BASE // 3, _triple, 0)

    @pl.when(wid < _EXTRA_TILES)
    def _():
        tb = 0
        pltpu.make_async_copy(feat_hbm.at[cv[tb]], gb[tb], gs[tb]).wait()
        pltpu.make_async_copy(
            vals_hbm.at[pl.ds(0, _CH)], vv[tb].at[pl.ds(0, _CH)],
            vs[tb]).wait()

        @plsc.parallel_loop(0, _CH, 1, unroll=4)
        def _scale_tail(e):
            sc_ = jnp.full((16,), vv[tb][pl.ds(e, 16)][0], jnp.float32)
            for j in range(_D // 16):
                sl = pl.ds(j * 16, 16)
                gb[tb][e, sl] = gb[tb][e, sl] * sc_

        pltpu.make_async_copy(rows_hbm.at[pl.ds(0, _CH)],
                              rv[tb], rs[tb]).wait()
        pltpu.async_copy(gb[tb], acc_sh.at[rv[tb]], ss[tb], add=True)

    for b in range(3):
        pltpu.make_async_copy(gb[b], acc_sh.at[rv[b]], ss[b]).wait()

    plsc.subcore_barrier()
    for i in range(_RCOPIES):
        r0 = row0 + i * _RCH
        pltpu.async_copy(acc_sh.at[pl.ds(r0, _RCH)],
                         out_hbm.at[c, pl.ds(r0, _RCH)], gsem0)
    for i in range(_RCOPIES):
        pltpu.make_async_copy(acc_sh.at[pl.ds(row0, _RCH)],
                              out_hbm.at[c, pl.ds(row0, _RCH)], gsem0).wait()


_sc_spmm = functools.partial(
    pl.kernel,
    out_type=jax.ShapeDtypeStruct((_NC, _NUM_OUT, _D), jnp.float32),
    mesh=plsc.VectorSubcoreMesh(core_axis_name="c", subcore_axis_name="s"),
    scratch_types=(
        [pltpu.VMEM((_CH,), jnp.int32)] * 3
        + [pltpu.VMEM((_CH + 16,), jnp.float32)] * 3
        + [pltpu.VMEM((_CH,), jnp.int32)] * 3
        + [pltpu.VMEM((_CH, _D), jnp.float32)] * 3
        + [pltpu.VMEM_SHARED((_NUM_OUT, _D), jnp.float32)]
        + [pltpu.SemaphoreType.DMA] * 15
    ),
)(_sc_body)


def _sum2_body(p_ref, o_ref):
    o_ref[...] = p_ref[0] + p_ref[1]


def _sum_partials(partials):
    blk = 2000
    return pl.pallas_call(
        _sum2_body,
        grid=(_NUM_OUT // blk,),
        in_specs=[pl.BlockSpec((_NC, blk, _D), lambda i: (0, i, 0))],
        out_specs=pl.BlockSpec((blk, _D), lambda i: (i, 0)),
        out_shape=jax.ShapeDtypeStruct((_NUM_OUT, _D), jnp.float32),
    )(partials)


@jax.jit
def kernel(simplex_features, boundary_indices, boundary_values):
    rows = boundary_indices[0].astype(jnp.int32)
    cols = boundary_indices[1].astype(jnp.int32)
    vals = boundary_values.astype(jnp.float32)
    partials = _sc_spmm(simplex_features, rows, cols, vals)
    return _sum_partials(partials)
